# baseline (speedup 1.0000x reference)
import functools

import jax
import jax.numpy as jnp
from jax import lax
from jax.experimental import pallas as pl
from jax.experimental.pallas import tpu as pltpu
from jax.experimental.pallas import tpu_sc as plsc

N = 10000
E = 320000
IN_DIM = 128
HID = 64
OUT = 64

NP_ = 10240
NW = 32
CHUNK = 128
CH = 80
EPT = CH * CHUNK
E_PAD = NW * EPT
NBUF = 4
ROWS_PER_TILE = NP_ // 16

_mesh = plsc.VectorSubcoreMesh(core_axis_name="c", subcore_axis_name="s")


def _deg_body(dst_hbm, out_hbm, didx, hist):
    cid = lax.axis_index("c")
    sid = lax.axis_index("s")
    w = cid * 16 + sid

    pltpu.sync_copy(dst_hbm.at[w], didx)

    zeros16 = jnp.zeros((16,), jnp.float32)

    def zero_body(i, carry):
        hist[i, :] = zeros16
        return carry

    lax.fori_loop(0, NP_ // 16, zero_body, 0)

    ones16 = jnp.ones((16,), jnp.float32)

    def acc_body(i, carry):
        idxv = didx[pl.ds(i * 16, 16)]
        row = lax.shift_right_logical(idxv, 4)
        col = lax.bitwise_and(idxv, 15)
        plsc.addupdate_scatter(hist, [row, col], ones16)
        return carry

    lax.fori_loop(0, EPT // 16, acc_body, 0)

    pltpu.sync_copy(hist, out_hbm.at[w])


_deg_kernel = functools.partial(
    pl.kernel,
    out_type=jax.ShapeDtypeStruct((NW, NP_ // 16, 16), jnp.float32),
    mesh=_mesh,
    compiler_params=pltpu.CompilerParams(needs_layout_passes=False),
    scratch_types=[
        pltpu.VMEM((EPT,), jnp.int32),
        pltpu.VMEM((NP_ // 16, 16), jnp.float32),
    ],
)(_deg_body)


def _prop_body(g_hbm, src_hbm, dst_hbm, z_hbm, out_hbm, sidx, didx, rows, accum,
               *gsems):
    cid = lax.axis_index("c")
    sid = lax.axis_index("s")
    w = cid * 16 + sid

    pltpu.sync_copy(src_hbm.at[w], sidx)
    pltpu.sync_copy(dst_hbm.at[w], didx)
    descs = [pltpu.async_copy(g_hbm.at[sidx.at[b]], rows.at[b], gsems[b])
             for b in range(NBUF)]
    pltpu.sync_copy(z_hbm.at[pl.ds(sid * ROWS_PER_TILE, ROWS_PER_TILE)],
                    accum.at[pl.ds(sid * ROWS_PER_TILE, ROWS_PER_TILE)])
    plsc.subcore_barrier()

    def body(j0, carry):
        for b in range(NBUF):
            j = j0 + b
            pltpu.make_async_copy(g_hbm.at[sidx.at[j]], rows.at[b],
                                  gsems[b]).wait()
            pltpu.sync_copy(rows.at[b], accum.at[didx.at[j]], add=True)

            @pl.when(j + NBUF < CH)
            def _():
                pltpu.async_copy(g_hbm.at[sidx.at[j + NBUF]], rows.at[b],
                                 gsems[b])
        return carry

    lax.fori_loop(0, CH // NBUF, lambda i, c: body(i * NBUF, c), 0)
    plsc.subcore_barrier()

    pltpu.sync_copy(accum.at[pl.ds(sid * ROWS_PER_TILE, ROWS_PER_TILE)],
                    out_hbm.at[cid, pl.ds(sid * ROWS_PER_TILE, ROWS_PER_TILE)])


_prop_kernel = functools.partial(
    pl.kernel,
    out_type=jax.ShapeDtypeStruct((2, NP_, HID), jnp.float32),
    mesh=_mesh,
    compiler_params=pltpu.CompilerParams(needs_layout_passes=False,
                                         use_tc_tiling_on_sc=False),
    scratch_types=[
        pltpu.VMEM((CH, CHUNK), jnp.int32),
        pltpu.VMEM((CH, CHUNK), jnp.int32),
        pltpu.VMEM((NBUF, CHUNK, HID), jnp.float32),
        pltpu.VMEM_SHARED((NP_, HID), jnp.float32),
    ] + [pltpu.SemaphoreType.DMA] * NBUF,
)(_prop_body)


def _dinv_block(degt):
    dsum = jnp.sum(degt, axis=1, keepdims=True)
    dinv = lax.rsqrt(dsum + 1.0)
    rows = lax.broadcasted_iota(jnp.int32, (NP_, 1), 0)
    return jnp.where(rows < N, dinv, 0.0)


def _tc1_body(x_ref, wet_ref, be_ref, w1t_ref, degt_ref, h_ref, g1_ref):
    h = jnp.dot(x_ref[...], wet_ref[...], preferred_element_type=jnp.float32)
    h = jnp.maximum(h + be_ref[...], 0.0)
    h_ref[...] = h
    dinv = _dinv_block(degt_ref[...])
    hw = jnp.dot(h, w1t_ref[...], preferred_element_type=jnp.float32)
    g1_ref[...] = hw * dinv


def _tc2_body(pa_ref, pb_ref, g1_ref, degt_ref, b1_ref, w2t_ref, h1_ref, g2_ref):
    dinv = _dinv_block(degt_ref[...])
    s = pa_ref[...] + pb_ref[...] + g1_ref[...]
    h1 = jnp.maximum(s * dinv + b1_ref[...], 0.0)
    h1_ref[...] = h1
    hw = jnp.dot(h1, w2t_ref[...], preferred_element_type=jnp.float32)
    g2_ref[...] = hw * dinv


def _tc3_body(pa_ref, pb_ref, g2_ref, degt_ref, b2_ref, h_ref, h1_ref,
              wca_ref, wcb_ref, wcc_ref, bc_ref, out_ref):
    dinv = _dinv_block(degt_ref[...])
    s = pa_ref[...] + pb_ref[...] + g2_ref[...]
    h2 = jnp.maximum(s * dinv + b2_ref[...], 0.0)
    out = jnp.dot(h_ref[...], wca_ref[...], preferred_element_type=jnp.float32)
    out = out + jnp.dot(h1_ref[...], wcb_ref[...], preferred_element_type=jnp.float32)
    out = out + jnp.dot(h2, wcc_ref[...], preferred_element_type=jnp.float32)
    out_ref[...] = out + bc_ref[...]


def _tc_call(body, n_out):
    return pl.pallas_call(
        body,
        out_shape=[jax.ShapeDtypeStruct((NP_, HID), jnp.float32)] * n_out,
    )


def kernel(x, edge_index, W_embed, b_embed, W1, b1, W2, b2, Wc, bc):
    f32 = jnp.float32
    src = edge_index[0]
    dst = edge_index[1]
    pad = E_PAD - E
    srcp = jnp.concatenate([src, jnp.full((pad,), N, jnp.int32)])
    dstp = jnp.concatenate([dst, jnp.full((pad,), N, jnp.int32)])
    src3 = srcp.reshape(NW, CH, CHUNK)
    dst3 = dstp.reshape(NW, CH, CHUNK)
    dstf = dstp.reshape(NW, EPT)

    xp = jnp.zeros((NP_, IN_DIM), f32).at[:N].set(x)
    zeros2d = jnp.zeros((NP_, HID), f32)

    wet = W_embed.T.astype(f32)
    w1t = W1.T.astype(f32)
    w2t = W2.T.astype(f32)
    wca = Wc[:, :HID].T.astype(f32)
    wcb = Wc[:, HID:2 * HID].T.astype(f32)
    wcc = Wc[:, 2 * HID:].T.astype(f32)
    be = b_embed.reshape(1, HID)
    b1r = b1.reshape(1, HID)
    b2r = b2.reshape(1, HID)
    bcr = bc.reshape(1, HID)

    degp = _deg_kernel(dstf).reshape(NW, NP_)
    degt = degp.T

    h, g1 = _tc_call(_tc1_body, 2)(xp, wet, be, w1t, degt)

    p1 = _prop_kernel(g1, src3, dst3, zeros2d)

    h1, g2 = _tc_call(_tc2_body, 2)(p1[0], p1[1], g1, degt, b1r, w2t)

    p2 = _prop_kernel(g2, src3, dst3, zeros2d)

    (out,) = _tc_call(_tc3_body, 1)(p2[0], p2[1], g2, degt, b2r, h, h1,
                                    wca, wcb, wcc, bcr)
    return out[:N]


# speedup vs baseline: 2.7630x; 2.7630x over previous
import functools

import jax
import jax.numpy as jnp
from jax import lax
from jax.experimental import pallas as pl
from jax.experimental.pallas import tpu as pltpu
from jax.experimental.pallas import tpu_sc as plsc

N = 10000
E = 320000
IN_DIM = 128
HID = 64
OUT = 64

NP_ = 10240
NW = 32
CHUNK = 128
CH = 80
EPT = CH * CHUNK
E_PAD = NW * EPT
NBUF = 4
ROWS_PER_TILE = NP_ // 16

_mesh = plsc.VectorSubcoreMesh(core_axis_name="c", subcore_axis_name="s")


def _deg_body(dst_hbm, out_hbm, didx, hist):
    cid = lax.axis_index("c")
    sid = lax.axis_index("s")
    w = cid * 16 + sid

    pltpu.sync_copy(dst_hbm.at[w], didx)

    zeros16 = jnp.zeros((16,), jnp.float32)

    def zero_body(i, carry):
        hist[i, :] = zeros16
        return carry

    lax.fori_loop(0, NP_ // 16, zero_body, 0)

    ones16 = jnp.ones((16,), jnp.float32)

    def acc_body(i, carry):
        idxv = didx[pl.ds(i * 16, 16)]
        row = lax.shift_right_logical(idxv, 4)
        col = lax.bitwise_and(idxv, 15)
        plsc.addupdate_scatter(hist, [row, col], ones16)
        return carry

    lax.fori_loop(0, EPT // 16, acc_body, 0)

    pltpu.sync_copy(hist, out_hbm.at[w])


_deg_kernel = functools.partial(
    pl.kernel,
    out_type=jax.ShapeDtypeStruct((NW, NP_ // 16, 16), jnp.float32),
    mesh=_mesh,
    compiler_params=pltpu.CompilerParams(needs_layout_passes=False),
    scratch_types=[
        pltpu.VMEM((EPT,), jnp.int32),
        pltpu.VMEM((NP_ // 16, 16), jnp.float32),
    ],
)(_deg_body)


def _prop_body(g_hbm, src_hbm, dst_hbm, z_hbm, out_hbm, sidx, didx, rows, accum,
               *gsems):
    cid = lax.axis_index("c")
    sid = lax.axis_index("s")
    w = cid * 16 + sid

    pltpu.sync_copy(src_hbm.at[w], sidx)
    pltpu.sync_copy(dst_hbm.at[w], didx)
    descs = [pltpu.async_copy(g_hbm.at[sidx.at[b]], rows.at[b], gsems[b])
             for b in range(NBUF)]
    pltpu.sync_copy(z_hbm.at[pl.ds(sid * ROWS_PER_TILE, ROWS_PER_TILE)],
                    accum.at[pl.ds(sid * ROWS_PER_TILE, ROWS_PER_TILE)])
    plsc.subcore_barrier()

    def body(j0, carry):
        for b in range(NBUF):
            j = j0 + b
            pltpu.make_async_copy(g_hbm.at[sidx.at[j]], rows.at[b],
                                  gsems[b]).wait()
            pltpu.sync_copy(rows.at[b], accum.at[didx.at[j]], add=True)

            @pl.when(j + NBUF < CH)
            def _():
                pltpu.async_copy(g_hbm.at[sidx.at[j + NBUF]], rows.at[b],
                                 gsems[b])
        return carry

    lax.fori_loop(0, CH // NBUF, lambda i, c: body(i * NBUF, c), 0)
    plsc.subcore_barrier()

    pltpu.sync_copy(accum.at[pl.ds(sid * ROWS_PER_TILE, ROWS_PER_TILE)],
                    out_hbm.at[cid, pl.ds(sid * ROWS_PER_TILE, ROWS_PER_TILE)])


_prop_kernel = functools.partial(
    pl.kernel,
    out_type=jax.ShapeDtypeStruct((2, NP_, HID), jnp.float32),
    mesh=_mesh,
    compiler_params=pltpu.CompilerParams(needs_layout_passes=False,
                                         use_tc_tiling_on_sc=False),
    scratch_types=[
        pltpu.VMEM((CH, CHUNK), jnp.int32),
        pltpu.VMEM((CH, CHUNK), jnp.int32),
        pltpu.VMEM((NBUF, CHUNK, HID), jnp.float32),
        pltpu.VMEM_SHARED((NP_, HID), jnp.float32),
    ] + [pltpu.SemaphoreType.DMA] * NBUF,
)(_prop_body)


def _dinv_block(degt):
    dsum = jnp.sum(degt, axis=1, keepdims=True)
    dinv = lax.rsqrt(dsum + 1.0)
    rows = lax.broadcasted_iota(jnp.int32, (NP_, 1), 0)
    return jnp.where(rows < N, dinv, 0.0)


def _tc1_body(x_ref, wet_ref, be_ref, w1t_ref, degt_ref, h_ref, g1_ref):
    h = jnp.dot(x_ref[...], wet_ref[...], preferred_element_type=jnp.float32)
    h = jnp.maximum(h + be_ref[...], 0.0)
    h_ref[...] = h
    dinv = _dinv_block(degt_ref[...])
    hw = jnp.dot(h, w1t_ref[...], preferred_element_type=jnp.float32)
    g1_ref[...] = hw * dinv


def _tc2_body(pa_ref, pb_ref, g1_ref, degt_ref, b1_ref, w2t_ref, h1_ref, g2_ref):
    dinv = _dinv_block(degt_ref[...])
    s = pa_ref[...] + pb_ref[...] + g1_ref[...]
    h1 = jnp.maximum(s * dinv + b1_ref[...], 0.0)
    h1_ref[...] = h1
    hw = jnp.dot(h1, w2t_ref[...], preferred_element_type=jnp.float32)
    g2_ref[...] = hw * dinv


def _tc3_body(pa_ref, pb_ref, g2_ref, degt_ref, b2_ref, h_ref, h1_ref,
              wca_ref, wcb_ref, wcc_ref, bc_ref, out_ref):
    dinv = _dinv_block(degt_ref[...])
    s = pa_ref[...] + pb_ref[...] + g2_ref[...]
    h2 = jnp.maximum(s * dinv + b2_ref[...], 0.0)
    out = jnp.dot(h_ref[...], wca_ref[...], preferred_element_type=jnp.float32)
    out = out + jnp.dot(h1_ref[...], wcb_ref[...], preferred_element_type=jnp.float32)
    out = out + jnp.dot(h2, wcc_ref[...], preferred_element_type=jnp.float32)
    out_ref[...] = out + bc_ref[...]


def _tc_call(body, n_out):
    return pl.pallas_call(
        body,
        out_shape=[jax.ShapeDtypeStruct((NP_, HID), jnp.float32)] * n_out,
    )


def kernel(x, edge_index, W_embed, b_embed, W1, b1, W2, b2, Wc, bc):
    f32 = jnp.float32
    src = edge_index[0]
    dst = edge_index[1]
    pad = E_PAD - E
    pad_idx = N + jnp.arange(pad, dtype=jnp.int32) % (NP_ - N)
    srcp = jnp.concatenate([src, pad_idx])
    dstp = jnp.concatenate([dst, pad_idx])
    src3 = srcp.reshape(NW, CH, CHUNK)
    dst3 = dstp.reshape(NW, CH, CHUNK)
    dstf = dstp.reshape(NW, EPT)

    xp = jnp.zeros((NP_, IN_DIM), f32).at[:N].set(x)
    zeros2d = jnp.zeros((NP_, HID), f32)

    wet = W_embed.T.astype(f32)
    w1t = W1.T.astype(f32)
    w2t = W2.T.astype(f32)
    wca = Wc[:, :HID].T.astype(f32)
    wcb = Wc[:, HID:2 * HID].T.astype(f32)
    wcc = Wc[:, 2 * HID:].T.astype(f32)
    be = b_embed.reshape(1, HID)
    b1r = b1.reshape(1, HID)
    b2r = b2.reshape(1, HID)
    bcr = bc.reshape(1, HID)

    degp = _deg_kernel(dstf).reshape(NW, NP_)
    degt = degp.T

    h, g1 = _tc_call(_tc1_body, 2)(xp, wet, be, w1t, degt)

    p1 = _prop_kernel(g1, src3, dst3, zeros2d)

    h1, g2 = _tc_call(_tc2_body, 2)(p1[0], p1[1], g1, degt, b1r, w2t)

    p2 = _prop_kernel(g2, src3, dst3, zeros2d)

    (out,) = _tc_call(_tc3_body, 1)(p2[0], p2[1], g2, degt, b2r, h, h1,
                                    wca, wcb, wcc, bcr)
    return out[:N]


# speedup vs baseline: 2.9869x; 1.0810x over previous
import functools

import jax
import jax.numpy as jnp
from jax import lax
from jax.experimental import pallas as pl
from jax.experimental.pallas import tpu as pltpu
from jax.experimental.pallas import tpu_sc as plsc

N = 10000
E = 320000
IN_DIM = 128
HID = 64
OUT = 64

NP_ = 10240
NW = 32
CHUNK = 128
CH = 80
EPT = CH * CHUNK
E_PAD = NW * EPT
NBUF = 4
ROWS_PER_TILE = NP_ // 16

_mesh = plsc.VectorSubcoreMesh(core_axis_name="c", subcore_axis_name="s")


def _deg_body(dst_hbm, out_hbm, didx, hist):
    cid = lax.axis_index("c")
    sid = lax.axis_index("s")
    w = cid * 16 + sid

    pltpu.sync_copy(dst_hbm.at[w], didx)

    zeros16 = jnp.zeros((16,), jnp.float32)

    def zero_body(i, carry):
        hist[i, :] = zeros16
        return carry

    lax.fori_loop(0, NP_ // 16, zero_body, 0)

    ones16 = jnp.ones((16,), jnp.float32)

    def acc_body(i, carry):
        idxv = didx[pl.ds(i * 16, 16)]
        row = lax.shift_right_logical(idxv, 4)
        col = lax.bitwise_and(idxv, 15)
        plsc.addupdate_scatter(hist, [row, col], ones16)
        return carry

    lax.fori_loop(0, EPT // 16, acc_body, 0)

    pltpu.sync_copy(hist, out_hbm.at[w])


_deg_kernel = functools.partial(
    pl.kernel,
    out_type=jax.ShapeDtypeStruct((NW, NP_ // 16, 16), jnp.float32),
    mesh=_mesh,
    compiler_params=pltpu.CompilerParams(needs_layout_passes=False),
    scratch_types=[
        pltpu.VMEM((EPT,), jnp.int32),
        pltpu.VMEM((NP_ // 16, 16), jnp.float32),
    ],
)(_deg_body)


def _prop_body(g_hbm, src_hbm, dst_hbm, z_hbm, out_hbm, sidx, didx, rows, accum,
               *gsems):
    cid = lax.axis_index("c")
    sid = lax.axis_index("s")
    w = cid * 16 + sid

    pltpu.sync_copy(src_hbm.at[w], sidx)
    pltpu.sync_copy(dst_hbm.at[w], didx)
    descs = [pltpu.async_copy(g_hbm.at[sidx.at[b]], rows.at[b], gsems[b])
             for b in range(NBUF)]
    pltpu.sync_copy(z_hbm.at[pl.ds(sid * ROWS_PER_TILE, ROWS_PER_TILE)],
                    accum.at[pl.ds(sid * ROWS_PER_TILE, ROWS_PER_TILE)])
    plsc.subcore_barrier()

    def body(j0, carry):
        for b in range(NBUF):
            j = j0 + b
            pltpu.make_async_copy(g_hbm.at[sidx.at[j]], rows.at[b],
                                  gsems[b]).wait()
            pltpu.sync_copy(rows.at[b], accum.at[didx.at[j]], add=True)

            @pl.when(j + NBUF < CH)
            def _():
                pltpu.async_copy(g_hbm.at[sidx.at[j + NBUF]], rows.at[b],
                                 gsems[b])
        return carry

    lax.fori_loop(0, CH // NBUF, lambda i, c: body(i * NBUF, c), 0)
    plsc.subcore_barrier()

    pltpu.sync_copy(accum.at[pl.ds(sid * ROWS_PER_TILE, ROWS_PER_TILE)],
                    out_hbm.at[cid, pl.ds(sid * ROWS_PER_TILE, ROWS_PER_TILE)])


_prop_kernel = functools.partial(
    pl.kernel,
    out_type=jax.ShapeDtypeStruct((2, NP_, HID), jnp.float32),
    mesh=_mesh,
    compiler_params=pltpu.CompilerParams(needs_layout_passes=False,
                                         use_tc_tiling_on_sc=False),
    scratch_types=[
        pltpu.VMEM((CH, CHUNK), jnp.int32),
        pltpu.VMEM((CH, CHUNK), jnp.int32),
        pltpu.VMEM((NBUF, CHUNK, HID), jnp.float32),
        pltpu.VMEM_SHARED((NP_, HID), jnp.float32),
    ] + [pltpu.SemaphoreType.DMA] * NBUF,
)(_prop_body)


def _dinv_block(degt):
    dsum = jnp.sum(degt, axis=1, keepdims=True)
    dinv = lax.rsqrt(dsum + 1.0)
    rows = lax.broadcasted_iota(jnp.int32, (NP_, 1), 0)
    return jnp.where(rows < N, dinv, 0.0)


def _tc1_body(x_ref, wet_ref, be_ref, w1t_ref, degt_ref, h_ref, g1_ref):
    h = jnp.dot(x_ref[...], wet_ref[...], preferred_element_type=jnp.float32)
    h = jnp.maximum(h + be_ref[...], 0.0)
    h_ref[...] = h
    dinv = _dinv_block(degt_ref[...])
    hw = jnp.dot(h, w1t_ref[...], preferred_element_type=jnp.float32)
    g1_ref[...] = hw * dinv


def _tc2_body(p_ref, g1_ref, degt_ref, b1_ref, w2t_ref, h1_ref, g2_ref):
    dinv = _dinv_block(degt_ref[...])
    s = p_ref[0] + p_ref[1] + g1_ref[...]
    h1 = jnp.maximum(s * dinv + b1_ref[...], 0.0)
    h1_ref[...] = h1
    hw = jnp.dot(h1, w2t_ref[...], preferred_element_type=jnp.float32)
    g2_ref[...] = hw * dinv


def _tc3_body(p_ref, g2_ref, degt_ref, b2_ref, h_ref, h1_ref,
              wca_ref, wcb_ref, wcc_ref, bc_ref, out_ref):
    dinv = _dinv_block(degt_ref[...])
    s = p_ref[0] + p_ref[1] + g2_ref[...]
    h2 = jnp.maximum(s * dinv + b2_ref[...], 0.0)
    out = jnp.dot(h_ref[...], wca_ref[...], preferred_element_type=jnp.float32)
    out = out + jnp.dot(h1_ref[...], wcb_ref[...], preferred_element_type=jnp.float32)
    out = out + jnp.dot(h2, wcc_ref[...], preferred_element_type=jnp.float32)
    out_ref[...] = out + bc_ref[...]


def _tc_call(body, n_out):
    return pl.pallas_call(
        body,
        out_shape=[jax.ShapeDtypeStruct((NP_, HID), jnp.float32)] * n_out,
    )


def kernel(x, edge_index, W_embed, b_embed, W1, b1, W2, b2, Wc, bc):
    f32 = jnp.float32
    src = edge_index[0]
    dst = edge_index[1]
    pad = E_PAD - E
    pad_idx = N + jnp.arange(pad, dtype=jnp.int32) % (NP_ - N)
    srcp = jnp.concatenate([src, pad_idx])
    dstp = jnp.concatenate([dst, pad_idx])
    src3 = srcp.reshape(NW, CH, CHUNK)
    dst3 = dstp.reshape(NW, CH, CHUNK)
    dstf = dstp.reshape(NW, EPT)

    xp = jnp.zeros((NP_, IN_DIM), f32).at[:N].set(x)
    zeros2d = jnp.zeros((NP_, HID), f32)

    wet = W_embed.T.astype(f32)
    w1t = W1.T.astype(f32)
    w2t = W2.T.astype(f32)
    wca = Wc[:, :HID].T.astype(f32)
    wcb = Wc[:, HID:2 * HID].T.astype(f32)
    wcc = Wc[:, 2 * HID:].T.astype(f32)
    be = b_embed.reshape(1, HID)
    b1r = b1.reshape(1, HID)
    b2r = b2.reshape(1, HID)
    bcr = bc.reshape(1, HID)

    degp = _deg_kernel(dstf).reshape(NW, NP_)
    degt = degp.T

    h, g1 = _tc_call(_tc1_body, 2)(xp, wet, be, w1t, degt)

    p1 = _prop_kernel(g1, src3, dst3, zeros2d)

    h1, g2 = _tc_call(_tc2_body, 2)(p1, g1, degt, b1r, w2t)

    p2 = _prop_kernel(g2, src3, dst3, zeros2d)

    (out,) = _tc_call(_tc3_body, 1)(p2, g2, degt, b2r, h, h1,
                                    wca, wcb, wcc, bcr)
    return out[:N]


# speedup vs baseline: 3.0904x; 1.0347x over previous
import functools

import jax
import jax.numpy as jnp
from jax import lax
from jax.experimental import pallas as pl
from jax.experimental.pallas import tpu as pltpu
from jax.experimental.pallas import tpu_sc as plsc

N = 10000
E = 320000
IN_DIM = 128
HID = 64
OUT = 64

NA = 10240
NW = 32
CHUNK = 128
CH = 80
EPT = CH * CHUNK
E_PAD = NW * EPT
NBUF = 4
RPT = N // 16

_mesh = plsc.VectorSubcoreMesh(core_axis_name="c", subcore_axis_name="s")


_HR = NA // 16
_HCH = _HR // 128


def _deg_body(dst_hbm, out_hbm, didx, hist, iidx, hacc):
    cid = lax.axis_index("c")
    sid = lax.axis_index("s")
    w = cid * 16 + sid

    pltpu.sync_copy(dst_hbm.at[w], didx)

    zeros16 = jnp.zeros((16,), jnp.float32)

    def zero_body(i, carry):
        hist[i, :] = zeros16
        return carry

    lax.fori_loop(0, _HR, zero_body, 0)

    iota16 = lax.iota(jnp.int32, 16)
    for c in range(_HCH):
        for k in range(8):
            iidx[c, pl.ds(k * 16, 16)] = c * 128 + k * 16 + iota16
    pltpu.sync_copy(hist.at[pl.ds(sid * (_HR // 16), _HR // 16)],
                    hacc.at[pl.ds(sid * (_HR // 16), _HR // 16)])
    plsc.subcore_barrier()

    ones16 = jnp.ones((16,), jnp.float32)

    def acc_body(i, carry):
        idxv = didx[pl.ds(i * 16, 16)]
        row = lax.shift_right_logical(idxv, 4)
        col = lax.bitwise_and(idxv, 15)
        plsc.addupdate_scatter(hist, [row, col], ones16)
        return carry

    lax.fori_loop(0, EPT // 16, acc_body, 0)

    for c in range(_HCH):
        pltpu.sync_copy(hist.at[pl.ds(c * 128, 128)], hacc.at[iidx.at[c]],
                        add=True)
    plsc.subcore_barrier()

    pltpu.sync_copy(hacc.at[pl.ds(sid * (_HR // 16), _HR // 16)],
                    out_hbm.at[cid, pl.ds(sid * (_HR // 16), _HR // 16)])


_deg_kernel = functools.partial(
    pl.kernel,
    out_type=jax.ShapeDtypeStruct((2, _HR, 16), jnp.float32),
    mesh=_mesh,
    compiler_params=pltpu.CompilerParams(needs_layout_passes=False,
                                         use_tc_tiling_on_sc=False),
    scratch_types=[
        pltpu.VMEM((EPT,), jnp.int32),
        pltpu.VMEM((_HR, 16), jnp.float32),
        pltpu.VMEM((_HCH, 128), jnp.int32),
        pltpu.VMEM_SHARED((_HR, 16), jnp.float32),
    ],
)(_deg_body)


def _prop_body(g_hbm, src_hbm, dst_hbm, out_hbm, sidx, didx, rows, zbuf, accum,
               *gsems):
    cid = lax.axis_index("c")
    sid = lax.axis_index("s")
    w = cid * 16 + sid

    pltpu.sync_copy(src_hbm.at[w], sidx)
    pltpu.sync_copy(dst_hbm.at[w], didx)
    for b in range(NBUF):
        pltpu.async_copy(g_hbm.at[sidx.at[b]], rows.at[b], gsems[b])

    zeros16 = jnp.zeros((16,), jnp.float32)

    def zero_body(i, carry):
        r = i // 4
        zbuf[r, pl.ds((i % 4) * 16, 16)] = zeros16
        return carry

    lax.fori_loop(0, 125 * 4, zero_body, 0)
    for r in range(5):
        pltpu.sync_copy(zbuf.at[pl.ds(0, 125)],
                        accum.at[pl.ds(sid * RPT + r * 125, 125)])
    plsc.subcore_barrier()

    def body(j0, carry):
        for b in range(NBUF):
            j = j0 + b
            pltpu.make_async_copy(g_hbm.at[sidx.at[j]], rows.at[b],
                                  gsems[b]).wait()
            pltpu.sync_copy(rows.at[b], accum.at[didx.at[j]], add=True)

            @pl.when(j + NBUF < CH)
            def _():
                pltpu.async_copy(g_hbm.at[sidx.at[j + NBUF]], rows.at[b],
                                 gsems[b])
        return carry

    lax.fori_loop(0, CH // NBUF, lambda i, c: body(i * NBUF, c), 0)
    plsc.subcore_barrier()

    pltpu.sync_copy(accum.at[pl.ds(sid * RPT, RPT)],
                    out_hbm.at[cid, pl.ds(sid * RPT, RPT)])


_prop_kernel = functools.partial(
    pl.kernel,
    out_type=jax.ShapeDtypeStruct((2, N, HID), jnp.float32),
    mesh=_mesh,
    compiler_params=pltpu.CompilerParams(needs_layout_passes=False,
                                         use_tc_tiling_on_sc=False),
    scratch_types=[
        pltpu.VMEM((CH, CHUNK), jnp.int32),
        pltpu.VMEM((CH, CHUNK), jnp.int32),
        pltpu.VMEM((NBUF, CHUNK, HID), jnp.float32),
        pltpu.VMEM((128, HID), jnp.float32),
        pltpu.VMEM_SHARED((NA, HID), jnp.float32),
    ] + [pltpu.SemaphoreType.DMA] * NBUF,
)(_prop_body)


RB = 1000
_GRID = N // RB


def _tc1_body(x_ref, wet_ref, be_ref, w1t_ref, dinv_ref, h_ref, g1_ref):
    h = jnp.dot(x_ref[...], wet_ref[...], preferred_element_type=jnp.float32)
    h = jnp.maximum(h + be_ref[...], 0.0)
    h_ref[...] = h
    hw = jnp.dot(h, w1t_ref[...], preferred_element_type=jnp.float32)
    g1_ref[...] = hw * dinv_ref[...]


def _tc2_body(p_ref, g1_ref, dinv_ref, b1_ref, w2t_ref, h1_ref, g2_ref):
    dinv = dinv_ref[...]
    s = p_ref[0] + p_ref[1] + g1_ref[...]
    h1 = jnp.maximum(s * dinv + b1_ref[...], 0.0)
    h1_ref[...] = h1
    hw = jnp.dot(h1, w2t_ref[...], preferred_element_type=jnp.float32)
    g2_ref[...] = hw * dinv


def _tc3_body(p_ref, g2_ref, dinv_ref, b2_ref, h_ref, h1_ref,
              wca_ref, wcb_ref, wcc_ref, bc_ref, out_ref):
    s = p_ref[0] + p_ref[1] + g2_ref[...]
    h2 = jnp.maximum(s * dinv_ref[...] + b2_ref[...], 0.0)
    out = jnp.dot(h_ref[...], wca_ref[...], preferred_element_type=jnp.float32)
    out = out + jnp.dot(h1_ref[...], wcb_ref[...], preferred_element_type=jnp.float32)
    out = out + jnp.dot(h2, wcc_ref[...], preferred_element_type=jnp.float32)
    out_ref[...] = out + bc_ref[...]


def _row_spec(cols):
    return pl.BlockSpec((RB, cols), lambda i: (i, 0))


def _full_spec(shape):
    return pl.BlockSpec(shape, lambda i: tuple(0 for _ in shape))


_P_SPEC = pl.BlockSpec((2, RB, HID), lambda i: (0, i, 0))
_OUT_ROW = jax.ShapeDtypeStruct((N, HID), jnp.float32)

_tc1_call = pl.pallas_call(
    _tc1_body,
    grid=(_GRID,),
    in_specs=[_row_spec(IN_DIM), _full_spec((IN_DIM, HID)),
              _full_spec((1, HID)), _full_spec((HID, HID)),
              _row_spec(HID)],
    out_specs=[_row_spec(HID)] * 2,
    out_shape=[_OUT_ROW] * 2,
)

_tc2_call = pl.pallas_call(
    _tc2_body,
    grid=(_GRID,),
    in_specs=[_P_SPEC, _row_spec(HID), _row_spec(HID),
              _full_spec((1, HID)), _full_spec((HID, HID))],
    out_specs=[_row_spec(HID)] * 2,
    out_shape=[_OUT_ROW] * 2,
)

_tc3_call = pl.pallas_call(
    _tc3_body,
    grid=(_GRID,),
    in_specs=[_P_SPEC, _row_spec(HID), _row_spec(HID), _full_spec((1, HID)),
              _row_spec(HID), _row_spec(HID),
              _full_spec((HID, HID)), _full_spec((HID, HID)),
              _full_spec((HID, HID)), _full_spec((1, HID))],
    out_specs=[_row_spec(HID)],
    out_shape=[_OUT_ROW],
)


def kernel(x, edge_index, W_embed, b_embed, W1, b1, W2, b2, Wc, bc):
    f32 = jnp.float32
    src = edge_index[0]
    dst = edge_index[1]
    pad = E_PAD - E
    arp = jnp.arange(pad, dtype=jnp.int32)
    srcp = jnp.concatenate([src, arp % N])
    dstp = jnp.concatenate([dst, N + arp % (NA - N)])
    src3 = srcp.reshape(NW, CH, CHUNK)
    dst3 = dstp.reshape(NW, CH, CHUNK)
    dstf = dstp.reshape(NW, EPT)

    wet = W_embed.T.astype(f32)
    w1t = W1.T.astype(f32)
    w2t = W2.T.astype(f32)
    wca = Wc[:, :HID].T.astype(f32)
    wcb = Wc[:, HID:2 * HID].T.astype(f32)
    wcc = Wc[:, 2 * HID:].T.astype(f32)
    be = b_embed.reshape(1, HID)
    b1r = b1.reshape(1, HID)
    b2r = b2.reshape(1, HID)
    bcr = bc.reshape(1, HID)

    degc = _deg_kernel(dstf)
    degf = (degc[0] + degc[1]).reshape(NA)[:N]
    dinv = jnp.broadcast_to(lax.rsqrt(degf + 1.0)[:, None], (N, HID))

    h, g1 = _tc1_call(x, wet, be, w1t, dinv)

    p1 = _prop_kernel(g1, src3, dst3)

    h1, g2 = _tc2_call(p1, g1, dinv, b1r, w2t)

    p2 = _prop_kernel(g2, src3, dst3)

    (out,) = _tc3_call(p2, g2, dinv, b2r, h, h1, wca, wcb, wcc, bcr)
    return out


# speedup vs baseline: 3.0930x; 1.0008x over previous
import functools

import jax
import jax.numpy as jnp
from jax import lax
from jax.experimental import pallas as pl
from jax.experimental.pallas import tpu as pltpu
from jax.experimental.pallas import tpu_sc as plsc

N = 10000
E = 320000
IN_DIM = 128
HID = 64
OUT = 64

NA = 10240
NW = 32
CHUNK = 128
CH = 80
EPT = CH * CHUNK
E_PAD = NW * EPT
NBUF = 4
RPT = N // 16

_mesh = plsc.VectorSubcoreMesh(core_axis_name="c", subcore_axis_name="s")


_HR = NA // 16
_HCH = _HR // 128


def _deg_body(dst_hbm, out_hbm, didx, hist, iidx, hacc):
    cid = lax.axis_index("c")
    sid = lax.axis_index("s")
    w = cid * 16 + sid

    pltpu.sync_copy(dst_hbm.at[w], didx)

    zeros16 = jnp.zeros((16,), jnp.float32)

    def zero_body(i, carry):
        hist[i, :] = zeros16
        return carry

    lax.fori_loop(0, _HR, zero_body, 0)

    iota16 = lax.iota(jnp.int32, 16)
    for c in range(_HCH):
        for k in range(8):
            iidx[c, pl.ds(k * 16, 16)] = c * 128 + k * 16 + iota16
    pltpu.sync_copy(hist.at[pl.ds(sid * (_HR // 16), _HR // 16)],
                    hacc.at[pl.ds(sid * (_HR // 16), _HR // 16)])
    plsc.subcore_barrier()

    ones16 = jnp.ones((16,), jnp.float32)

    def acc_body(i, carry):
        idxv = didx[pl.ds(i * 16, 16)]
        row = lax.shift_right_logical(idxv, 4)
        col = lax.bitwise_and(idxv, 15)
        plsc.addupdate_scatter(hist, [row, col], ones16)
        return carry

    lax.fori_loop(0, EPT // 16, acc_body, 0)

    for c in range(_HCH):
        pltpu.sync_copy(hist.at[pl.ds(c * 128, 128)], hacc.at[iidx.at[c]],
                        add=True)
    plsc.subcore_barrier()

    pltpu.sync_copy(hacc.at[pl.ds(sid * (_HR // 16), _HR // 16)],
                    out_hbm.at[cid, pl.ds(sid * (_HR // 16), _HR // 16)])


_deg_kernel = functools.partial(
    pl.kernel,
    out_type=jax.ShapeDtypeStruct((2, _HR, 16), jnp.float32),
    mesh=_mesh,
    compiler_params=pltpu.CompilerParams(needs_layout_passes=False,
                                         use_tc_tiling_on_sc=False),
    scratch_types=[
        pltpu.VMEM((EPT,), jnp.int32),
        pltpu.VMEM((_HR, 16), jnp.float32),
        pltpu.VMEM((_HCH, 128), jnp.int32),
        pltpu.VMEM_SHARED((_HR, 16), jnp.float32),
    ],
)(_deg_body)


def _prop_body(g_hbm, src_hbm, dst_hbm, out_hbm, sidx, didx, rows, zbuf, accum,
               *gsems):
    cid = lax.axis_index("c")
    sid = lax.axis_index("s")
    w = cid * 16 + sid

    pltpu.sync_copy(src_hbm.at[w], sidx)
    pltpu.sync_copy(dst_hbm.at[w], didx)
    for b in range(NBUF):
        pltpu.async_copy(g_hbm.at[sidx.at[b]], rows.at[b], gsems[b])

    zeros16 = jnp.zeros((16,), jnp.float32)

    def zero_body(i, carry):
        r = i // 4
        zbuf[r, pl.ds((i % 4) * 16, 16)] = zeros16
        return carry

    lax.fori_loop(0, 125 * 4, zero_body, 0)
    for r in range(5):
        pltpu.sync_copy(zbuf.at[pl.ds(0, 125)],
                        accum.at[pl.ds(sid * RPT + r * 125, 125)])
    plsc.subcore_barrier()

    def body(j0, carry):
        for b in range(NBUF):
            j = j0 + b
            pltpu.make_async_copy(g_hbm.at[sidx.at[j]], rows.at[b],
                                  gsems[b]).wait()
            pltpu.sync_copy(rows.at[b], accum.at[didx.at[j]], add=True)

            @pl.when(j + NBUF < CH)
            def _():
                pltpu.async_copy(g_hbm.at[sidx.at[j + NBUF]], rows.at[b],
                                 gsems[b])
        return carry

    lax.fori_loop(0, CH // NBUF, lambda i, c: body(i * NBUF, c), 0)
    plsc.subcore_barrier()

    pltpu.sync_copy(accum.at[pl.ds(sid * RPT, RPT)],
                    out_hbm.at[cid, pl.ds(sid * RPT, RPT)])


_prop_kernel = functools.partial(
    pl.kernel,
    out_type=jax.ShapeDtypeStruct((2, N, HID), jnp.float32),
    mesh=_mesh,
    compiler_params=pltpu.CompilerParams(needs_layout_passes=False,
                                         use_tc_tiling_on_sc=False),
    scratch_types=[
        pltpu.VMEM((CH, CHUNK), jnp.int32),
        pltpu.VMEM((CH, CHUNK), jnp.int32),
        pltpu.VMEM((NBUF, CHUNK, HID), jnp.float32),
        pltpu.VMEM((128, HID), jnp.float32),
        pltpu.VMEM_SHARED((NA, HID), jnp.float32),
    ] + [pltpu.SemaphoreType.DMA] * NBUF,
)(_prop_body)


RB = 1000
_GRID = N // RB


def _tc1_body(x_ref, wet_ref, be_ref, w1t_ref, h_ref, hw_ref):
    h = jnp.dot(x_ref[...], wet_ref[...], preferred_element_type=jnp.float32)
    h = jnp.maximum(h + be_ref[...], 0.0)
    h_ref[...] = h
    hw_ref[...] = jnp.dot(h, w1t_ref[...], preferred_element_type=jnp.float32)


def _tc2_body(p_ref, g1_ref, dinv_ref, b1_ref, w2t_ref, h1_ref, g2_ref):
    dinv = dinv_ref[...]
    s = p_ref[0] + p_ref[1] + g1_ref[...]
    h1 = jnp.maximum(s * dinv + b1_ref[...], 0.0)
    h1_ref[...] = h1
    hw = jnp.dot(h1, w2t_ref[...], preferred_element_type=jnp.float32)
    g2_ref[...] = hw * dinv


def _tc3_body(p_ref, g2_ref, dinv_ref, b2_ref, h_ref, h1_ref,
              wca_ref, wcb_ref, wcc_ref, bc_ref, out_ref):
    s = p_ref[0] + p_ref[1] + g2_ref[...]
    h2 = jnp.maximum(s * dinv_ref[...] + b2_ref[...], 0.0)
    out = jnp.dot(h_ref[...], wca_ref[...], preferred_element_type=jnp.float32)
    out = out + jnp.dot(h1_ref[...], wcb_ref[...], preferred_element_type=jnp.float32)
    out = out + jnp.dot(h2, wcc_ref[...], preferred_element_type=jnp.float32)
    out_ref[...] = out + bc_ref[...]


def _row_spec(cols):
    return pl.BlockSpec((RB, cols), lambda i: (i, 0))


def _full_spec(shape):
    return pl.BlockSpec(shape, lambda i: tuple(0 for _ in shape))


_P_SPEC = pl.BlockSpec((2, RB, HID), lambda i: (0, i, 0))
_OUT_ROW = jax.ShapeDtypeStruct((N, HID), jnp.float32)

_tc1_call = pl.pallas_call(
    _tc1_body,
    grid=(_GRID,),
    in_specs=[_row_spec(IN_DIM), _full_spec((IN_DIM, HID)),
              _full_spec((1, HID)), _full_spec((HID, HID))],
    out_specs=[_row_spec(HID)] * 2,
    out_shape=[_OUT_ROW] * 2,
)

_tc2_call = pl.pallas_call(
    _tc2_body,
    grid=(_GRID,),
    in_specs=[_P_SPEC, _row_spec(HID), _row_spec(HID),
              _full_spec((1, HID)), _full_spec((HID, HID))],
    out_specs=[_row_spec(HID)] * 2,
    out_shape=[_OUT_ROW] * 2,
)

_tc3_call = pl.pallas_call(
    _tc3_body,
    grid=(_GRID,),
    in_specs=[_P_SPEC, _row_spec(HID), _row_spec(HID), _full_spec((1, HID)),
              _row_spec(HID), _row_spec(HID),
              _full_spec((HID, HID)), _full_spec((HID, HID)),
              _full_spec((HID, HID)), _full_spec((1, HID))],
    out_specs=[_row_spec(HID)],
    out_shape=[_OUT_ROW],
)


def kernel(x, edge_index, W_embed, b_embed, W1, b1, W2, b2, Wc, bc):
    f32 = jnp.float32
    src = edge_index[0]
    dst = edge_index[1]
    pad = E_PAD - E
    arp = jnp.arange(pad, dtype=jnp.int32)
    srcp = jnp.concatenate([src, arp % N])
    dstp = jnp.concatenate([dst, N + arp % (NA - N)])
    src3 = srcp.reshape(NW, CH, CHUNK)
    dst3 = dstp.reshape(NW, CH, CHUNK)
    dstf = dstp.reshape(NW, EPT)

    wet = W_embed.T.astype(f32)
    w1t = W1.T.astype(f32)
    w2t = W2.T.astype(f32)
    wca = Wc[:, :HID].T.astype(f32)
    wcb = Wc[:, HID:2 * HID].T.astype(f32)
    wcc = Wc[:, 2 * HID:].T.astype(f32)
    be = b_embed.reshape(1, HID)
    b1r = b1.reshape(1, HID)
    b2r = b2.reshape(1, HID)
    bcr = bc.reshape(1, HID)

    degc = _deg_kernel(dstf)

    h, hw1 = _tc1_call(x, wet, be, w1t)

    degf = (degc[0] + degc[1]).reshape(NA)[:N]
    dinv = jnp.broadcast_to(lax.rsqrt(degf + 1.0)[:, None], (N, HID))
    g1 = hw1 * dinv

    p1 = _prop_kernel(g1, src3, dst3)

    h1, g2 = _tc2_call(p1, g1, dinv, b1r, w2t)

    p2 = _prop_kernel(g2, src3, dst3)

    (out,) = _tc3_call(p2, g2, dinv, b2r, h, h1, wca, wcb, wcc, bcr)
    return out


# speedup vs baseline: 3.4999x; 1.1316x over previous
import functools

import jax
import jax.numpy as jnp
from jax import lax
from jax.experimental import pallas as pl
from jax.experimental.pallas import tpu as pltpu
from jax.experimental.pallas import tpu_sc as plsc

N = 10000
E = 320000
IN_DIM = 128
HID = 64
OUT = 64

NA = 10240
NW = 32
CHUNK = 128
CH = 80
EPT = CH * CHUNK
E_PAD = NW * EPT
NBUF = 4
RPT = N // 16

_mesh = plsc.VectorSubcoreMesh(core_axis_name="c", subcore_axis_name="s")


_HR = NA // 16
_HCH = _HR // 128


def _deg_body(dst_hbm, out_hbm, didx, hist, iidx, hacc):
    cid = lax.axis_index("c")
    sid = lax.axis_index("s")
    w = cid * 16 + sid

    pltpu.sync_copy(dst_hbm.at[w], didx)

    zeros16 = jnp.zeros((16,), jnp.float32)

    def zero_body(i, carry):
        hist[i, :] = zeros16
        return carry

    lax.fori_loop(0, _HR, zero_body, 0)

    iota16 = lax.iota(jnp.int32, 16)
    for c in range(_HCH):
        for k in range(8):
            iidx[c, pl.ds(k * 16, 16)] = c * 128 + k * 16 + iota16
    pltpu.sync_copy(hist.at[pl.ds(sid * (_HR // 16), _HR // 16)],
                    hacc.at[pl.ds(sid * (_HR // 16), _HR // 16)])
    plsc.subcore_barrier()

    ones16 = jnp.ones((16,), jnp.float32)

    def acc_body(i, carry):
        idxv = didx[pl.ds(i * 16, 16)]
        row = lax.shift_right_logical(idxv, 4)
        col = lax.bitwise_and(idxv, 15)
        plsc.addupdate_scatter(hist, [row, col], ones16)
        return carry

    lax.fori_loop(0, EPT // 16, acc_body, 0)

    for c in range(_HCH):
        pltpu.sync_copy(hist.at[pl.ds(c * 128, 128)], hacc.at[iidx.at[c]],
                        add=True)
    plsc.subcore_barrier()

    pltpu.sync_copy(hacc.at[pl.ds(sid * (_HR // 16), _HR // 16)],
                    out_hbm.at[cid, pl.ds(sid * (_HR // 16), _HR // 16)])


_deg_kernel = functools.partial(
    pl.kernel,
    out_type=jax.ShapeDtypeStruct((2, _HR, 16), jnp.float32),
    mesh=_mesh,
    compiler_params=pltpu.CompilerParams(needs_layout_passes=False,
                                         use_tc_tiling_on_sc=False),
    scratch_types=[
        pltpu.VMEM((EPT,), jnp.int32),
        pltpu.VMEM((_HR, 16), jnp.float32),
        pltpu.VMEM((_HCH, 128), jnp.int32),
        pltpu.VMEM_SHARED((_HR, 16), jnp.float32),
    ],
)(_deg_body)


def _prop_body(g_hbm, src_hbm, dst_hbm, out_hbm, sidx, didx, rows, zbuf, accum,
               *gsems):
    cid = lax.axis_index("c")
    sid = lax.axis_index("s")
    w = cid * 16 + sid

    pltpu.sync_copy(src_hbm.at[w], sidx)
    pltpu.sync_copy(dst_hbm.at[w], didx)
    for b in range(NBUF):
        pltpu.async_copy(g_hbm.at[sidx.at[b]], rows.at[b], gsems[b])

    zeros16 = jnp.zeros((16,), jnp.float32)

    def zero_body(i, carry):
        r = i // 4
        zbuf[r, pl.ds((i % 4) * 16, 16)] = zeros16
        return carry

    lax.fori_loop(0, 125 * 4, zero_body, 0)
    for r in range(5):
        pltpu.sync_copy(zbuf.at[pl.ds(0, 125)],
                        accum.at[pl.ds(sid * RPT + r * 125, 125)])
    plsc.subcore_barrier()

    def body(j0, carry):
        for b in range(NBUF):
            j = j0 + b
            pltpu.make_async_copy(g_hbm.at[sidx.at[j]], rows.at[b],
                                  gsems[b]).wait()
            pltpu.sync_copy(rows.at[b], accum.at[didx.at[j]], add=True)

            @pl.when(j + NBUF < CH)
            def _():
                pltpu.async_copy(g_hbm.at[sidx.at[j + NBUF]], rows.at[b],
                                 gsems[b])
        return carry

    lax.fori_loop(0, CH // NBUF, lambda i, c: body(i * NBUF, c), 0)
    plsc.subcore_barrier()

    pltpu.sync_copy(accum.at[pl.ds(sid * RPT, RPT)],
                    out_hbm.at[cid, pl.ds(sid * RPT, RPT)])


_prop_kernel = functools.partial(
    pl.kernel,
    out_type=jax.ShapeDtypeStruct((2, N, HID), jnp.float32),
    mesh=_mesh,
    compiler_params=pltpu.CompilerParams(needs_layout_passes=False,
                                         use_tc_tiling_on_sc=False),
    scratch_types=[
        pltpu.VMEM((CH, CHUNK), jnp.int32),
        pltpu.VMEM((CH, CHUNK), jnp.int32),
        pltpu.VMEM((NBUF, CHUNK, HID), jnp.float32),
        pltpu.VMEM((128, HID), jnp.float32),
        pltpu.VMEM_SHARED((NA, HID), jnp.float32),
    ] + [pltpu.SemaphoreType.DMA] * NBUF,
)(_prop_body)


NH = N // 2
PW = 2 * HID
RB = 1000
_GRID = NH // RB


def _tc1_body(x_ref, wet_ref, be_ref, w1t_ref, h_ref, hw_ref):
    h = jnp.dot(x_ref[...], wet_ref[...], preferred_element_type=jnp.float32)
    h = jnp.maximum(h + be_ref[...], 0.0)
    h_ref[...] = h
    hw_ref[...] = jnp.dot(h, w1t_ref[...], preferred_element_type=jnp.float32)


def _tc2_body(p_ref, g1_ref, dinv_ref, b1_ref, w2t_ref, h1_ref, g2_ref):
    dinv = dinv_ref[...]
    s = p_ref[0] + p_ref[1] + g1_ref[...]
    h1 = jnp.maximum(s * dinv + b1_ref[...], 0.0)
    h1_ref[...] = h1
    hw = jnp.dot(h1, w2t_ref[...], preferred_element_type=jnp.float32)
    g2_ref[...] = hw * dinv


def _tc3_body(p_ref, g2_ref, dinv_ref, b2_ref, h_ref, h1_ref,
              wca_ref, wcb_ref, wcc_ref, bc_ref, out_ref):
    s = p_ref[0] + p_ref[1] + g2_ref[...]
    h2 = jnp.maximum(s * dinv_ref[...] + b2_ref[...], 0.0)
    out = jnp.dot(h_ref[...], wca_ref[...], preferred_element_type=jnp.float32)
    out = out + jnp.dot(h1_ref[...], wcb_ref[...], preferred_element_type=jnp.float32)
    out = out + jnp.dot(h2, wcc_ref[...], preferred_element_type=jnp.float32)
    out_ref[...] = out + bc_ref[...]


def _row_spec(cols):
    return pl.BlockSpec((RB, cols), lambda i: (i, 0))


def _full_spec(shape):
    return pl.BlockSpec(shape, lambda i: tuple(0 for _ in shape))


_P_SPEC = pl.BlockSpec((2, RB, PW), lambda i: (0, i, 0))
_OUT_ROW = jax.ShapeDtypeStruct((NH, PW), jnp.float32)

_tc1_call = pl.pallas_call(
    _tc1_body,
    grid=(_GRID,),
    in_specs=[_row_spec(2 * IN_DIM), _full_spec((2 * IN_DIM, PW)),
              _full_spec((1, PW)), _full_spec((PW, PW))],
    out_specs=[_row_spec(PW)] * 2,
    out_shape=[_OUT_ROW] * 2,
)

_tc2_call = pl.pallas_call(
    _tc2_body,
    grid=(_GRID,),
    in_specs=[_P_SPEC, _row_spec(PW), _row_spec(PW),
              _full_spec((1, PW)), _full_spec((PW, PW))],
    out_specs=[_row_spec(PW)] * 2,
    out_shape=[_OUT_ROW] * 2,
)

_tc3_call = pl.pallas_call(
    _tc3_body,
    grid=(_GRID,),
    in_specs=[_P_SPEC, _row_spec(PW), _row_spec(PW), _full_spec((1, PW)),
              _row_spec(PW), _row_spec(PW),
              _full_spec((PW, PW)), _full_spec((PW, PW)),
              _full_spec((PW, PW)), _full_spec((1, PW))],
    out_specs=[_row_spec(PW)],
    out_shape=[_OUT_ROW],
)


def _bd(w):
    z = jnp.zeros_like(w)
    return jnp.concatenate(
        [jnp.concatenate([w, z], axis=1), jnp.concatenate([z, w], axis=1)],
        axis=0)


def kernel(x, edge_index, W_embed, b_embed, W1, b1, W2, b2, Wc, bc):
    f32 = jnp.float32
    pad = E_PAD - E
    arp = jnp.arange(pad, dtype=jnp.int32)
    dstp = jnp.concatenate([edge_index[1], N + arp % (NA - N)])
    dst3 = dstp.reshape(NW, CH, CHUNK)
    dstf = dstp.reshape(NW, EPT)
    ei2 = lax.optimization_barrier(edge_index)
    srcp = jnp.concatenate([ei2[0], arp % N])
    src3 = srcp.reshape(NW, CH, CHUNK)

    wet2 = _bd(W_embed.T.astype(f32))
    w1t2 = _bd(W1.T.astype(f32))
    w2t2 = _bd(W2.T.astype(f32))
    wca2 = _bd(Wc[:, :HID].T.astype(f32))
    wcb2 = _bd(Wc[:, HID:2 * HID].T.astype(f32))
    wcc2 = _bd(Wc[:, 2 * HID:].T.astype(f32))
    be2 = jnp.concatenate([b_embed, b_embed]).reshape(1, PW)
    b12 = jnp.concatenate([b1, b1]).reshape(1, PW)
    b22 = jnp.concatenate([b2, b2]).reshape(1, PW)
    bc2 = jnp.concatenate([bc, bc]).reshape(1, PW)
    xp = x.reshape(NH, 2 * IN_DIM)

    degc = _deg_kernel(dstf)

    h, hw1 = _tc1_call(xp, wet2, be2, w1t2)

    degf = (degc[0] + degc[1]).reshape(NA)[:N]
    dinv = jnp.broadcast_to(
        lax.rsqrt(degf + 1.0).reshape(NH, 2, 1), (NH, 2, HID)).reshape(NH, PW)
    g1 = hw1 * dinv

    p1 = _prop_kernel(g1.reshape(N, HID), src3, dst3)

    h1, g2 = _tc2_call(p1.reshape(2, NH, PW), g1, dinv, b12, w2t2)

    p2 = _prop_kernel(g2.reshape(N, HID), src3, dst3)

    (out,) = _tc3_call(p2.reshape(2, NH, PW), g2, dinv, b22, h, h1,
                       wca2, wcb2, wcc2, bc2)
    return out.reshape(N, HID)


# speedup vs baseline: 3.7827x; 1.0808x over previous
import functools

import jax
import jax.numpy as jnp
from jax import lax
from jax.experimental import pallas as pl
from jax.experimental.pallas import tpu as pltpu
from jax.experimental.pallas import tpu_sc as plsc

N = 10000
E = 320000
IN_DIM = 128
HID = 64
OUT = 64

NA = 10240
NW = 32
CHUNK = 128
CH = 80
EPT = CH * CHUNK
E_PAD = NW * EPT
NBUF = 4
RPT = N // 16

_mesh = plsc.VectorSubcoreMesh(core_axis_name="c", subcore_axis_name="s")


_HR = NA // 16
_HCH = _HR // 128


def _deg_body(dst_hbm, out_hbm, didx, hist, iidx, hacc):
    cid = lax.axis_index("c")
    sid = lax.axis_index("s")
    w = cid * 16 + sid

    pltpu.sync_copy(dst_hbm.at[w], didx)

    zeros16 = jnp.zeros((16,), jnp.float32)

    def zero_body(i, carry):
        hist[i, :] = zeros16
        return carry

    lax.fori_loop(0, _HR, zero_body, 0)

    iota16 = lax.iota(jnp.int32, 16)
    for c in range(_HCH):
        for k in range(8):
            iidx[c, pl.ds(k * 16, 16)] = c * 128 + k * 16 + iota16
    pltpu.sync_copy(hist.at[pl.ds(sid * (_HR // 16), _HR // 16)],
                    hacc.at[pl.ds(sid * (_HR // 16), _HR // 16)])
    plsc.subcore_barrier()

    ones16 = jnp.ones((16,), jnp.float32)

    def acc_body(i, carry):
        idxv = didx[pl.ds(i * 16, 16)]
        row = lax.shift_right_logical(idxv, 4)
        col = lax.bitwise_and(idxv, 15)
        plsc.addupdate_scatter(hist, [row, col], ones16)
        return carry

    lax.fori_loop(0, EPT // 16, acc_body, 0)

    for c in range(_HCH):
        pltpu.sync_copy(hist.at[pl.ds(c * 128, 128)], hacc.at[iidx.at[c]],
                        add=True)
    plsc.subcore_barrier()

    pltpu.sync_copy(hacc.at[pl.ds(sid * (_HR // 16), _HR // 16)],
                    out_hbm.at[cid, pl.ds(sid * (_HR // 16), _HR // 16)])


_deg_kernel = functools.partial(
    pl.kernel,
    out_type=jax.ShapeDtypeStruct((2, _HR, 16), jnp.float32),
    mesh=_mesh,
    compiler_params=pltpu.CompilerParams(needs_layout_passes=False,
                                         use_tc_tiling_on_sc=False),
    scratch_types=[
        pltpu.VMEM((EPT,), jnp.int32),
        pltpu.VMEM((_HR, 16), jnp.float32),
        pltpu.VMEM((_HCH, 128), jnp.int32),
        pltpu.VMEM_SHARED((_HR, 16), jnp.float32),
    ],
)(_deg_body)


def _prop_body(g_hbm, src_hbm, dst_hbm, out_hbm, sidx, didx, rows, zbuf, accum,
               *gsems):
    cid = lax.axis_index("c")
    sid = lax.axis_index("s")
    w = cid * 16 + sid

    pltpu.sync_copy(src_hbm.at[w], sidx)
    pltpu.sync_copy(dst_hbm.at[w], didx)
    for b in range(NBUF):
        pltpu.async_copy(g_hbm.at[sidx.at[b]], rows.at[b], gsems[b])

    zeros16 = jnp.zeros((16,), jnp.float32)

    def zero_body(i, carry):
        r = i // 4
        zbuf[r, pl.ds((i % 4) * 16, 16)] = zeros16
        return carry

    lax.fori_loop(0, 125 * 4, zero_body, 0)
    for r in range(5):
        pltpu.sync_copy(zbuf.at[pl.ds(0, 125)],
                        accum.at[pl.ds(sid * RPT + r * 125, 125)])
    plsc.subcore_barrier()

    def body(j0, carry):
        for b in range(NBUF):
            j = j0 + b
            pltpu.make_async_copy(g_hbm.at[sidx.at[j]], rows.at[b],
                                  gsems[b]).wait()
            pltpu.sync_copy(rows.at[b], accum.at[didx.at[j]], add=True)

            @pl.when(j + NBUF < CH)
            def _():
                pltpu.async_copy(g_hbm.at[sidx.at[j + NBUF]], rows.at[b],
                                 gsems[b])
        return carry

    lax.fori_loop(0, CH // NBUF, lambda i, c: body(i * NBUF, c), 0)
    plsc.subcore_barrier()

    pltpu.sync_copy(accum.at[pl.ds(sid * RPT, RPT)],
                    out_hbm.at[cid, pl.ds(sid * RPT, RPT)])


_prop_kernel = functools.partial(
    pl.kernel,
    out_type=jax.ShapeDtypeStruct((2, N, HID), jnp.float32),
    mesh=_mesh,
    compiler_params=pltpu.CompilerParams(needs_layout_passes=False,
                                         use_tc_tiling_on_sc=False),
    scratch_types=[
        pltpu.VMEM((CH, CHUNK), jnp.int32),
        pltpu.VMEM((CH, CHUNK), jnp.int32),
        pltpu.VMEM((NBUF, CHUNK, HID), jnp.float32),
        pltpu.VMEM((128, HID), jnp.float32),
        pltpu.VMEM_SHARED((NA, HID), jnp.float32),
    ] + [pltpu.SemaphoreType.DMA] * NBUF,
)(_prop_body)


NH = N // 2
PW = 2 * HID
RB = 1000
_GRID = NH // RB


def _tc1_body(x_ref, wet_ref, be_ref, w1t_ref, h_ref, hw_ref):
    h = jnp.dot(x_ref[...], wet_ref[...], preferred_element_type=jnp.float32)
    h = jnp.maximum(h + be_ref[...], 0.0)
    h_ref[...] = h
    hw_ref[...] = jnp.dot(h, w1t_ref[...], preferred_element_type=jnp.float32)


def _tc2_body(p_ref, g1_ref, dinv_ref, b1_ref, w2t_ref, h1_ref, g2_ref):
    dinv = dinv_ref[...]
    s = p_ref[0] + p_ref[1] + g1_ref[...]
    h1 = jnp.maximum(s * dinv + b1_ref[...], 0.0)
    h1_ref[...] = h1
    hw = jnp.dot(h1, w2t_ref[...], preferred_element_type=jnp.float32)
    g2_ref[...] = hw * dinv


def _tc3_body(p_ref, g2_ref, dinv_ref, b2_ref, h_ref, h1_ref,
              wca_ref, wcb_ref, wcc_ref, bc_ref, out_ref):
    s = p_ref[0] + p_ref[1] + g2_ref[...]
    h2 = jnp.maximum(s * dinv_ref[...] + b2_ref[...], 0.0)
    out = jnp.dot(h_ref[...], wca_ref[...], preferred_element_type=jnp.float32)
    out = out + jnp.dot(h1_ref[...], wcb_ref[...], preferred_element_type=jnp.float32)
    out = out + jnp.dot(h2, wcc_ref[...], preferred_element_type=jnp.float32)
    out_ref[...] = out + bc_ref[...]


def _row_spec(cols):
    return pl.BlockSpec((RB, cols), lambda i: (i, 0))


def _full_spec(shape):
    return pl.BlockSpec(shape, lambda i: tuple(0 for _ in shape))


_P_SPEC = pl.BlockSpec((2, RB, PW), lambda i: (0, i, 0))
_OUT_ROW = jax.ShapeDtypeStruct((NH, PW), jnp.float32)

_tc1_call = pl.pallas_call(
    _tc1_body,
    grid=(_GRID,),
    in_specs=[_row_spec(2 * IN_DIM), _full_spec((2 * IN_DIM, PW)),
              _full_spec((1, PW)), _full_spec((PW, PW))],
    out_specs=[_row_spec(PW)] * 2,
    out_shape=[_OUT_ROW] * 2,
)

_tc2_call = pl.pallas_call(
    _tc2_body,
    grid=(_GRID,),
    in_specs=[_P_SPEC, _row_spec(PW), _row_spec(PW),
              _full_spec((1, PW)), _full_spec((PW, PW))],
    out_specs=[_row_spec(PW)] * 2,
    out_shape=[_OUT_ROW] * 2,
)

_tc3_call = pl.pallas_call(
    _tc3_body,
    grid=(_GRID,),
    in_specs=[_P_SPEC, _row_spec(PW), _row_spec(PW), _full_spec((1, PW)),
              _row_spec(PW), _row_spec(PW),
              _full_spec((PW, PW)), _full_spec((PW, PW)),
              _full_spec((PW, PW)), _full_spec((1, PW))],
    out_specs=[_row_spec(PW)],
    out_shape=[_OUT_ROW],
)


def _bd(w):
    z = jnp.zeros_like(w)
    return jnp.concatenate(
        [jnp.concatenate([w, z], axis=1), jnp.concatenate([z, w], axis=1)],
        axis=0)


def kernel(x, edge_index, W_embed, b_embed, W1, b1, W2, b2, Wc, bc):
    f32 = jnp.float32
    src = edge_index[0]
    dst = edge_index[1]
    pad = E_PAD - E
    arp = jnp.arange(pad, dtype=jnp.int32)
    srcp = jnp.concatenate([src, arp % N])
    dstp = jnp.concatenate([dst, N + arp % (NA - N)])
    src3 = srcp.reshape(NW, CH, CHUNK)
    dst3 = dstp.reshape(NW, CH, CHUNK)
    dstf = dstp.reshape(NW, EPT)

    wet2 = _bd(W_embed.T.astype(f32))
    w1t2 = _bd(W1.T.astype(f32))
    w2t2 = _bd(W2.T.astype(f32))
    wca2 = _bd(Wc[:, :HID].T.astype(f32))
    wcb2 = _bd(Wc[:, HID:2 * HID].T.astype(f32))
    wcc2 = _bd(Wc[:, 2 * HID:].T.astype(f32))
    be2 = jnp.concatenate([b_embed, b_embed]).reshape(1, PW)
    b12 = jnp.concatenate([b1, b1]).reshape(1, PW)
    b22 = jnp.concatenate([b2, b2]).reshape(1, PW)
    bc2 = jnp.concatenate([bc, bc]).reshape(1, PW)
    xp = x.reshape(NH, 2 * IN_DIM)

    degc = _deg_kernel(dstf)

    h, hw1 = _tc1_call(xp, wet2, be2, w1t2)

    degf = (degc[0] + degc[1]).reshape(NA)[:N]
    dinv = jnp.broadcast_to(
        lax.rsqrt(degf + 1.0).reshape(NH, 2, 1), (NH, 2, HID)).reshape(NH, PW)
    g1 = hw1 * dinv

    p1 = _prop_kernel(g1.reshape(N, HID), src3, dst3)

    h1, g2 = _tc2_call(p1.reshape(2, NH, PW), g1, dinv, b12, w2t2)

    p2 = _prop_kernel(g2.reshape(N, HID), src3, dst3)

    (out,) = _tc3_call(p2.reshape(2, NH, PW), g2, dinv, b22, h, h1,
                       wca2, wcb2, wcc2, bc2)
    return out.reshape(N, HID)
